# baseline (speedup 1.0000x reference)
import jax
import jax.numpy as jnp
from jax.experimental import pallas as pl
from jax.experimental.pallas import tpu as pltpu


def _body(ar_ref, gate_ref, x_ref, emb_ref, o_ref):
    bi = pl.program_id(0)
    w = ar_ref[bi, 0]
    h = ar_ref[bi, 1]
    g = jnp.tanh(gate_ref[0])
    t = x_ref.shape[2]
    rows = []
    for ti in range(t):
        idx = (ti // h) * t + (ti % h)
        rows.append(emb_ref[idx, 0, :].reshape(1, -1))
    table = jnp.concatenate(rows, axis=0)
    tile_id = jax.lax.broadcasted_iota(jnp.int32, (t, 1), 0)
    scale = jnp.where(tile_id < w * h, g, jnp.zeros_like(g))
    table = (table * scale)[None, None]
    o_ref[...] = x_ref[...] + table


def kernel(x, ar, embedding, gate):
    b, t, n, w = x.shape
    xt = jnp.transpose(x, (0, 2, 1, 3))
    emb = embedding.reshape(t * t, 1, w)
    tb = 500
    ntb = pl.cdiv(n, tb)

    def x_map(bi, ni, ar_ref, gate_ref):
        return (bi, ni, 0, 0)

    def emb_map(bi, ni, ar_ref, gate_ref):
        return (0, 0, 0)

    grid_spec = pltpu.PrefetchScalarGridSpec(
        num_scalar_prefetch=2,
        grid=(b, ntb),
        in_specs=[
            pl.BlockSpec((1, tb, t, w), x_map),
            pl.BlockSpec((t * t, 1, w), emb_map),
        ],
        out_specs=pl.BlockSpec((1, tb, t, w), x_map),
    )
    res = pl.pallas_call(
        _body,
        grid_spec=grid_spec,
        out_shape=jax.ShapeDtypeStruct(xt.shape, x.dtype),
    )(ar, gate, xt, emb)
    return jnp.transpose(res, (0, 2, 1, 3))


# speedup vs baseline: 1.0109x; 1.0109x over previous
import jax
import jax.numpy as jnp
from jax.experimental import pallas as pl
from jax.experimental.pallas import tpu as pltpu


def _body(ar_ref, gate_ref, x_ref, emb_ref, o_ref):
    bi = pl.program_id(0)
    w = ar_ref[bi, 0]
    h = ar_ref[bi, 1]
    g = jnp.tanh(gate_ref[0])
    t = x_ref.shape[2]
    rows = []
    for ti in range(t):
        idx = (ti // h) * t + (ti % h)
        rows.append(emb_ref[idx, 0, :].reshape(1, -1))
    table = jnp.concatenate(rows, axis=0)
    tile_id = jax.lax.broadcasted_iota(jnp.int32, (t, 1), 0)
    scale = jnp.where(tile_id < w * h, g, jnp.zeros_like(g))
    table = (table * scale)[None, None]
    o_ref[...] = x_ref[...] + table


def kernel(x, ar, embedding, gate):
    b, t, n, w = x.shape
    xt = jnp.transpose(x, (0, 2, 1, 3))
    emb = embedding.reshape(t * t, 1, w)
    tb = 488
    ntb = pl.cdiv(n, tb)

    def x_map(bi, ni, ar_ref, gate_ref):
        return (bi, ni, 0, 0)

    def emb_map(bi, ni, ar_ref, gate_ref):
        return (0, 0, 0)

    grid_spec = pltpu.PrefetchScalarGridSpec(
        num_scalar_prefetch=2,
        grid=(b, ntb),
        in_specs=[
            pl.BlockSpec((1, tb, t, w), x_map),
            pl.BlockSpec((t * t, 1, w), emb_map),
        ],
        out_specs=pl.BlockSpec((1, tb, t, w), x_map),
    )
    res = pl.pallas_call(
        _body,
        grid_spec=grid_spec,
        out_shape=jax.ShapeDtypeStruct(xt.shape, x.dtype),
    )(ar, gate, xt, emb)
    return jnp.transpose(res, (0, 2, 1, 3))


# speedup vs baseline: 1.0133x; 1.0023x over previous
import jax
import jax.numpy as jnp
from jax.experimental import pallas as pl
from jax.experimental.pallas import tpu as pltpu


def _body(ar_ref, gate_ref, x_ref, emb_ref, o_ref):
    bi = pl.program_id(0)
    w = ar_ref[bi, 0]
    h = ar_ref[bi, 1]
    g = jnp.tanh(gate_ref[0])
    t = x_ref.shape[2]
    rows = []
    for ti in range(t):
        idx = (ti // h) * t + (ti % h)
        rows.append(emb_ref[idx, 0, :].reshape(1, -1))
    table = jnp.concatenate(rows, axis=0)
    tile_id = jax.lax.broadcasted_iota(jnp.int32, (t, 1), 0)
    scale = jnp.where(tile_id < w * h, g, jnp.zeros_like(g))
    table = (table * scale)[None, None]
    o_ref[...] = x_ref[...] + table


def kernel(x, ar, embedding, gate):
    b, t, n, w = x.shape
    xt = jnp.transpose(x, (0, 2, 1, 3))
    emb = embedding.reshape(t * t, 1, w)
    tb = 480
    ntb = pl.cdiv(n, tb)

    def x_map(bi, ni, ar_ref, gate_ref):
        return (bi, ni, 0, 0)

    def emb_map(bi, ni, ar_ref, gate_ref):
        return (0, 0, 0)

    grid_spec = pltpu.PrefetchScalarGridSpec(
        num_scalar_prefetch=2,
        grid=(b, ntb),
        in_specs=[
            pl.BlockSpec((1, tb, t, w), x_map),
            pl.BlockSpec((t * t, 1, w), emb_map),
        ],
        out_specs=pl.BlockSpec((1, tb, t, w), x_map),
    )
    res = pl.pallas_call(
        _body,
        grid_spec=grid_spec,
        out_shape=jax.ShapeDtypeStruct(xt.shape, x.dtype),
        compiler_params=pltpu.CompilerParams(
            dimension_semantics=("parallel", "arbitrary")),
    )(ar, gate, xt, emb)
    return jnp.transpose(res, (0, 2, 1, 3))
